# baseline (speedup 1.0000x reference)
import functools

import jax
import jax.numpy as jnp
from jax import lax
from jax.experimental import pallas as pl
from jax.experimental.pallas import tpu as pltpu
from jax.experimental.pallas import tpu_sc as plsc

_INFO = plsc.get_sparse_core_info()
_NC = _INFO.num_cores
_NS = _INFO.num_subcores
_NW = _NC * _NS


def _embed_lookup(idx_flat, table, *, chunk):
    n = idx_flat.shape[0]
    d = table.shape[1]
    per_w = n // _NW
    n_chunks = per_w // chunk
    n_pairs = n_chunks // 2
    mesh = plsc.VectorSubcoreMesh(core_axis_name="c", subcore_axis_name="s")

    @functools.partial(
        pl.kernel,
        mesh=mesh,
        compiler_params=pltpu.CompilerParams(use_tc_tiling_on_sc=False),
        out_type=jax.ShapeDtypeStruct((n, d), jnp.float32),
        scratch_types=[
            pltpu.VMEM((per_w,), jnp.int32),
            pltpu.VMEM((chunk, d), jnp.float32),
            pltpu.VMEM((chunk, d), jnp.float32),
            pltpu.SemaphoreType.DMA,
            pltpu.SemaphoreType.DMA,
            pltpu.SemaphoreType.DMA,
            pltpu.SemaphoreType.DMA,
        ],
    )
    def k(idx_hbm, table_hbm, out_hbm, idx_v, buf0, buf1, g0, g1, s0, s1):
        wid = lax.axis_index("s") * _NC + lax.axis_index("c")
        w_base = wid * per_w
        pltpu.sync_copy(idx_hbm.at[pl.ds(w_base, per_w)], idx_v)

        def gather(c, buf, sem):
            pltpu.async_copy(table_hbm.at[idx_v.at[pl.ds(c * chunk, chunk)]], buf, sem)

        def gather_wait(c, buf, sem):
            pltpu.make_async_copy(
                table_hbm.at[idx_v.at[pl.ds(c * chunk, chunk)]], buf, sem
            ).wait()

        def store(c, buf, sem):
            pltpu.async_copy(buf, out_hbm.at[pl.ds(w_base + c * chunk, chunk)], sem)

        def store_wait(c, buf, sem):
            pltpu.make_async_copy(
                buf, out_hbm.at[pl.ds(w_base + c * chunk, chunk)], sem
            ).wait()

        gather(0, buf0, g0)
        gather(1, buf1, g1)

        def pair(p, _):
            c0 = 2 * p
            gather_wait(c0, buf0, g0)
            store(c0, buf0, s0)
            gather_wait(c0 + 1, buf1, g1)
            store(c0 + 1, buf1, s1)
            store_wait(c0, buf0, s0)
            gather(c0 + 2, buf0, g0)
            store_wait(c0 + 1, buf1, s1)
            gather(c0 + 3, buf1, g1)
            return 0

        lax.fori_loop(0, n_pairs - 1, pair, 0)

        c0 = 2 * (n_pairs - 1)
        gather_wait(c0, buf0, g0)
        store(c0, buf0, s0)
        gather_wait(c0 + 1, buf1, g1)
        store(c0 + 1, buf1, s1)
        store_wait(c0, buf0, s0)
        store_wait(c0 + 1, buf1, s1)

    return k(idx_flat, table)


def kernel(input_ids, table):
    b, h = input_ids.shape
    d = table.shape[1]
    idx_flat = input_ids.reshape(b * h).astype(jnp.int32)
    out = _embed_lookup(idx_flat, table, chunk=800)
    return out.reshape(b, h, d)
